# baseline (speedup 1.0000x reference)
import functools

import jax
import jax.numpy as jnp
from jax import lax
from jax.experimental import pallas as pl
from jax.experimental.pallas import tpu as pltpu
from jax.experimental.pallas import tpu_sc as plsc

N = 10000
E = 320000
D = 128

_NC = 2
_NS = 16
_NW = _NC * _NS
_EPW = E // _NW
_CH = 80
_NCHUNK = _EPW // _CH
_NRB = 3
_NIB = 5
_NITER = 135
_RPT = 624
_ZR = 48


def _sc_segment_sum(x, edges):
    mesh = plsc.VectorSubcoreMesh(core_axis_name="c", subcore_axis_name="s")

    @functools.partial(
        pl.kernel,
        out_type=jax.ShapeDtypeStruct((_NC, N, D), jnp.float32),
        mesh=mesh,
        scratch_types=[
            pltpu.VMEM((_NIB, _CH), jnp.int32),
            pltpu.VMEM((_NIB, _CH), jnp.int32),
            pltpu.VMEM((_NRB, _CH, D), jnp.float32),
            pltpu.VMEM((_ZR, D), jnp.float32),
            pltpu.VMEM_SHARED((N, D), jnp.float32),
            pltpu.SemaphoreType.DMA((_NIB,)),
            pltpu.SemaphoreType.DMA((_NRB,)),
            pltpu.SemaphoreType.DMA((_NRB,)),
        ],
    )
    def seg(x_hbm, edges_hbm, out_hbm, sidx_v, didx_v, rows_v, zbuf,
            u_sh, isem, gsem, ssem):
        c = lax.axis_index("c")
        s = lax.axis_index("s")
        wid = s * _NC + c

        ebase = wid * _EPW

        def _iload(k, b):
            pltpu.async_copy(
                edges_hbm.at[pl.ds(ebase + k * _CH, _CH)], sidx_v.at[b],
                isem.at[b])
            pltpu.async_copy(
                edges_hbm.at[pl.ds(E + ebase + k * _CH, _CH)], didx_v.at[b],
                isem.at[b])

        def _iwait(b):
            pltpu.make_async_copy(
                edges_hbm.at[pl.ds(0, _CH)], sidx_v.at[b], isem.at[b]).wait()
            pltpu.make_async_copy(
                edges_hbm.at[pl.ds(0, _CH)], didx_v.at[b], isem.at[b]).wait()

        def _gwait(b):
            pltpu.make_async_copy(
                x_hbm.at[pl.ds(0, _CH)], rows_v.at[b], gsem.at[b]).wait()

        def _swait(b):
            pltpu.make_async_copy(
                x_hbm.at[pl.ds(0, _CH)], rows_v.at[b], ssem.at[b]).wait()

        for b in range(3):
            _iload(b, b)

        zero16 = jnp.zeros((16,), jnp.float32)

        def _zrow(i, carry):
            for j in range(D // 16):
                zbuf[i, pl.ds(j * 16, 16)] = zero16
            return carry

        lax.fori_loop(0, _ZR, _zrow, 0)

        zcopies = []
        for r in range(_RPT // _ZR):
            zcopies.append(pltpu.async_copy(
                zbuf, u_sh.at[pl.ds(s * _RPT + r * _ZR, _ZR)], ssem.at[0]))

        @pl.when(s == _NS - 1)
        def _():
            pltpu.async_copy(
                zbuf.at[pl.ds(0, N - _NS * _RPT)],
                u_sh.at[pl.ds(_NS * _RPT, N - _NS * _RPT)],
                ssem.at[0]).wait()

        _iwait(0)
        pltpu.async_copy(x_hbm.at[sidx_v.at[0]], rows_v.at[0], gsem.at[0])
        for cp in zcopies:
            cp.wait()
        plsc.subcore_barrier()

        def _group(g, carry):
            for j in range(15):
                kb = g * 15 + j
                b3 = j % _NRB
                b5 = j % _NIB
                bs = (j + 1) % _NRB
                bi = (j + 3) % _NIB
                bw = (j + 1) % _NIB

                @pl.when((kb >= 2) & (kb < _NCHUNK + 2))
                def _():
                    _swait(bs)

                @pl.when(kb + 3 < _NCHUNK)
                def _():
                    _iload(kb + 3, bi)

                @pl.when(kb + 1 < _NCHUNK)
                def _():
                    _iwait(bw)
                    pltpu.async_copy(
                        x_hbm.at[sidx_v.at[bw]], rows_v.at[bs], gsem.at[bs])

                @pl.when(kb < _NCHUNK)
                def _():
                    _gwait(b3)
                    pltpu.async_copy(
                        rows_v.at[b3], u_sh.at[didx_v.at[b5]], ssem.at[b3],
                        add=True)
            return carry

        lax.fori_loop(0, _NITER // 15, _group, 0)
        plsc.subcore_barrier()

        pltpu.sync_copy(
            u_sh.at[pl.ds(s * _RPT, _RPT)],
            out_hbm.at[c, pl.ds(s * _RPT, _RPT)],
        )

        @pl.when(s == _NS - 1)
        def _():
            pltpu.sync_copy(
                u_sh.at[pl.ds(_NS * _RPT, N - _NS * _RPT)],
                out_hbm.at[c, pl.ds(_NS * _RPT, N - _NS * _RPT)],
            )

    return seg(x, edges)


_BLK = 10000


def _tc_body(u2_ref, x_ref, w_ref, b_ref, g_ref, be_ref, o_ref):
    u = u2_ref[0] + u2_ref[1]
    h = lax.dot_general(u, w_ref[...], (((1,), (1,)), ((), ())),
                        preferred_element_type=jnp.float32)
    h = jnp.maximum(h + b_ref[...], 0.0) + x_ref[...]
    mu = jnp.mean(h, axis=-1, keepdims=True)
    var = jnp.mean((h - mu) ** 2, axis=-1, keepdims=True)
    o_ref[...] = (h - mu) * lax.rsqrt(var + 1e-5) * g_ref[...] + be_ref[...]


def _tc_fused(u2, x, w, b, gamma, beta):
    return pl.pallas_call(
        _tc_body,
        grid=(N // _BLK,),
        in_specs=[
            pl.BlockSpec((_NC, _BLK, D), lambda i: (0, i, 0)),
            pl.BlockSpec((_BLK, D), lambda i: (i, 0)),
            pl.BlockSpec((D, D), lambda i: (0, 0)),
            pl.BlockSpec((1, D), lambda i: (0, 0)),
            pl.BlockSpec((1, D), lambda i: (0, 0)),
            pl.BlockSpec((1, D), lambda i: (0, 0)),
        ],
        out_specs=pl.BlockSpec((_BLK, D), lambda i: (i, 0)),
        out_shape=jax.ShapeDtypeStruct((N, D), jnp.float32),
    )(u2, x, w, b.reshape(1, D), gamma.reshape(1, D), beta.reshape(1, D))


def kernel(x, edge_index, W, b, gamma, beta):
    edges = edge_index.astype(jnp.int32).reshape(2 * E)
    u2 = _sc_segment_sum(x, edges)
    return _tc_fused(u2, x, W, b, gamma, beta)


# speedup vs baseline: 1.0207x; 1.0207x over previous
import functools

import jax
import jax.numpy as jnp
from jax import lax
from jax.experimental import pallas as pl
from jax.experimental.pallas import tpu as pltpu
from jax.experimental.pallas import tpu_sc as plsc

N = 10000
E = 320000
D = 128

_NC = 2
_NS = 16
_NW = _NC * _NS
_EPW = E // _NW
_CH = 80
_NCHUNK = _EPW // _CH
_NRB = 3
_NIB = 5
_NITER = 135
_RPT = 624
_ZR = 48


def _sc_segment_sum(x, edges):
    mesh = plsc.VectorSubcoreMesh(core_axis_name="c", subcore_axis_name="s")

    @functools.partial(
        pl.kernel,
        out_type=jax.ShapeDtypeStruct((_NC, N, D), jnp.float32),
        mesh=mesh,
        scratch_types=[
            pltpu.VMEM((_NIB, _CH), jnp.int32),
            pltpu.VMEM((_NIB, _CH), jnp.int32),
            pltpu.VMEM((_NRB, _CH, D), jnp.float32),
            pltpu.VMEM((_ZR, D), jnp.float32),
            pltpu.VMEM_SHARED((N, D), jnp.float32),
            pltpu.SemaphoreType.DMA((_NIB,)),
            pltpu.SemaphoreType.DMA((_NRB,)),
            pltpu.SemaphoreType.DMA((_NRB,)),
        ],
    )
    def seg(x_hbm, edges_hbm, out_hbm, sidx_v, didx_v, rows_v, zbuf,
            u_sh, isem, gsem, ssem):
        c = lax.axis_index("c")
        s = lax.axis_index("s")
        wid = s * _NC + c

        ebase = wid * _EPW

        def _iload(k, b):
            pltpu.async_copy(
                edges_hbm.at[pl.ds(ebase + k * _CH, _CH)], sidx_v.at[b],
                isem.at[b])
            pltpu.async_copy(
                edges_hbm.at[pl.ds(E + ebase + k * _CH, _CH)], didx_v.at[b],
                isem.at[b])

        def _iwait(b):
            pltpu.make_async_copy(
                edges_hbm.at[pl.ds(0, _CH)], sidx_v.at[b], isem.at[b]).wait()
            pltpu.make_async_copy(
                edges_hbm.at[pl.ds(0, _CH)], didx_v.at[b], isem.at[b]).wait()

        def _gwait(b):
            pltpu.make_async_copy(
                x_hbm.at[pl.ds(0, _CH)], rows_v.at[b], gsem.at[b]).wait()

        def _swait(b):
            pltpu.make_async_copy(
                x_hbm.at[pl.ds(0, _CH)], rows_v.at[b], ssem.at[b]).wait()

        for b in range(3):
            _iload(b, b)

        zero16 = jnp.zeros((16,), jnp.float32)

        def _zrow(i, carry):
            for j in range(D // 16):
                zbuf[i, pl.ds(j * 16, 16)] = zero16
            return carry

        lax.fori_loop(0, _ZR, _zrow, 0)

        zcopies = []
        for r in range(_RPT // _ZR):
            zcopies.append(pltpu.async_copy(
                zbuf, u_sh.at[pl.ds(s * _RPT + r * _ZR, _ZR)], ssem.at[0]))

        @pl.when(s == _NS - 1)
        def _():
            pltpu.async_copy(
                zbuf.at[pl.ds(0, N - _NS * _RPT)],
                u_sh.at[pl.ds(_NS * _RPT, N - _NS * _RPT)],
                ssem.at[0]).wait()

        _iwait(0)
        pltpu.async_copy(x_hbm.at[sidx_v.at[0]], rows_v.at[0], gsem.at[0])
        for cp in zcopies:
            cp.wait()
        plsc.subcore_barrier()

        def _group(g, carry):
            for j in range(15):
                kb = g * 15 + j
                b3 = j % _NRB
                b5 = j % _NIB
                bs = (j + 1) % _NRB
                bi = (j + 3) % _NIB
                bw = (j + 1) % _NIB

                @pl.when((kb >= 2) & (kb < _NCHUNK + 2))
                def _():
                    _swait(bs)

                @pl.when(kb + 3 < _NCHUNK)
                def _():
                    _iload(kb + 3, bi)

                @pl.when(kb + 1 < _NCHUNK)
                def _():
                    _iwait(bw)
                    pltpu.async_copy(
                        x_hbm.at[sidx_v.at[bw]], rows_v.at[bs], gsem.at[bs])

                @pl.when(kb < _NCHUNK)
                def _():
                    _gwait(b3)
                    pltpu.async_copy(
                        rows_v.at[b3], u_sh.at[didx_v.at[b5]], ssem.at[b3],
                        add=True)
            return carry

        lax.fori_loop(0, _NITER // 15, _group, 0)
        plsc.subcore_barrier()

        pltpu.sync_copy(
            u_sh.at[pl.ds(s * _RPT, _RPT)],
            out_hbm.at[c, pl.ds(s * _RPT, _RPT)],
        )

        @pl.when(s == _NS - 1)
        def _():
            pltpu.sync_copy(
                u_sh.at[pl.ds(_NS * _RPT, N - _NS * _RPT)],
                out_hbm.at[c, pl.ds(_NS * _RPT, N - _NS * _RPT)],
            )

    return seg(x, edges)


_BLK = 5000


def _tc_body(u2_ref, x_ref, w_ref, b_ref, g_ref, be_ref, o_ref):
    u = u2_ref[0] + u2_ref[1]
    h = lax.dot_general(u, w_ref[...], (((1,), (1,)), ((), ())),
                        preferred_element_type=jnp.float32)
    h = jnp.maximum(h + b_ref[...], 0.0) + x_ref[...]
    mu = jnp.mean(h, axis=-1, keepdims=True)
    var = jnp.mean((h - mu) ** 2, axis=-1, keepdims=True)
    o_ref[...] = (h - mu) * lax.rsqrt(var + 1e-5) * g_ref[...] + be_ref[...]


def _tc_fused(u2, x, w, b, gamma, beta):
    return pl.pallas_call(
        _tc_body,
        grid=(N // _BLK,),
        in_specs=[
            pl.BlockSpec((_NC, _BLK, D), lambda i: (0, i, 0)),
            pl.BlockSpec((_BLK, D), lambda i: (i, 0)),
            pl.BlockSpec((D, D), lambda i: (0, 0)),
            pl.BlockSpec((1, D), lambda i: (0, 0)),
            pl.BlockSpec((1, D), lambda i: (0, 0)),
            pl.BlockSpec((1, D), lambda i: (0, 0)),
        ],
        out_specs=pl.BlockSpec((_BLK, D), lambda i: (i, 0)),
        out_shape=jax.ShapeDtypeStruct((N, D), jnp.float32),
    )(u2, x, w, b.reshape(1, D), gamma.reshape(1, D), beta.reshape(1, D))


def kernel(x, edge_index, W, b, gamma, beta):
    edges = edge_index.astype(jnp.int32).reshape(2 * E)
    u2 = _sc_segment_sum(x, edges)
    return _tc_fused(u2, x, W, b, gamma, beta)
